# Initial kernel scaffold; baseline (speedup 1.0000x reference)
import jax
import jax.numpy as jnp
from jax.experimental import pallas as pl


def kernel(x, codebook):
    raise NotImplementedError("write your pallas kernel here")


# speedup vs baseline: 6.6520x; 6.6520x over previous
import jax
import jax.numpy as jnp
from jax.experimental import pallas as pl
from jax.experimental.pallas import tpu as pltpu

M = 32
K = 256
D = 24
EMB = M * D


def _pq_kernel(x_ref, cb_ref, quant_ref, codes_ref):
    x = x_ref[:]
    codes_cols = []
    quant_cols = []
    for m in range(M):
        cb_m = cb_ref[m]
        x_m = x[:, m * D:(m + 1) * D]
        scores = jax.lax.dot_general(x_m, cb_m, (((1,), (1,)), ((), ())))
        codes_m = jnp.argmax(scores, axis=1).astype(jnp.int32)
        onehot = (jax.lax.broadcasted_iota(jnp.int32, scores.shape, 1)
                  == codes_m[:, None]).astype(jnp.float32)
        quant_m = jax.lax.dot_general(onehot, cb_m, (((1,), (0,)), ((), ())),
                                      precision=jax.lax.Precision.HIGHEST)
        codes_cols.append(codes_m[:, None])
        quant_cols.append(quant_m)
    quant_ref[:] = jnp.concatenate(quant_cols, axis=1)
    codes_ref[:] = jnp.concatenate(codes_cols, axis=1)


def kernel(x, codebook):
    B = x.shape[0]
    BT = 512
    grid = (B // BT,)
    quant, codes = pl.pallas_call(
        _pq_kernel,
        grid=grid,
        in_specs=[pl.BlockSpec((BT, EMB), lambda i: (i, 0)),
                  pl.BlockSpec((M, K, D), lambda i: (0, 0, 0))],
        out_specs=[pl.BlockSpec((BT, EMB), lambda i: (i, 0)),
                   pl.BlockSpec((BT, M), lambda i: (i, 0))],
        out_shape=(jax.ShapeDtypeStruct((B, EMB), jnp.float32),
                   jax.ShapeDtypeStruct((B, M), jnp.int32)),
        compiler_params=pltpu.CompilerParams(
            dimension_semantics=("parallel",)),
    )(x, codebook)
    return quant, codes
